# Initial kernel scaffold; baseline (speedup 1.0000x reference)
import functools

import jax
import jax.numpy as jnp
from jax import lax
from jax.experimental import pallas as pl
from jax.experimental.pallas import tpu as pltpu
from jax.experimental.pallas import tpu_sc as plsc

EMBED = 32
NC, NS = 2, 16
NW = NC * NS
CHUNK = 2048


@functools.lru_cache(maxsize=None)
def _make_gather(B: int):
    per_w = B // NW
    nch = per_w // CHUNK
    mesh = plsc.VectorSubcoreMesh(core_axis_name="c", subcore_axis_name="s")

    @functools.partial(
        pl.kernel,
        mesh=mesh,
        out_type=jax.ShapeDtypeStruct((B, EMBED), jnp.float32),
        scratch_types=[
            pltpu.VMEM((CHUNK,), jnp.int32),
            pltpu.VMEM((CHUNK, EMBED), jnp.float32),
            pltpu.SemaphoreType.DMA,
        ],
    )
    def k(idx_hbm, table_hbm, out_hbm, idx_v, rows_v, sem):
        wid = lax.axis_index("s") * NC + lax.axis_index("c")
        base_w = wid * per_w

        def body(g, carry):
            base = base_w + g * CHUNK
            pltpu.sync_copy(idx_hbm.at[pl.ds(base, CHUNK)], idx_v)
            pltpu.async_copy(table_hbm.at[idx_v], rows_v, sem).wait()
            pltpu.sync_copy(rows_v, out_hbm.at[pl.ds(base, CHUNK)])
            return carry

        lax.fori_loop(0, nch, body, 0)

    return k


def kernel(input_indices, table):
    bsz, hist = input_indices.shape
    B = bsz * hist
    idx = input_indices.reshape(B).astype(jnp.int32)
    step = NW * CHUNK
    B_pad = ((B + step - 1) // step) * step
    if B_pad != B:
        idx = jnp.pad(idx, (0, B_pad - B))
    out = _make_gather(B_pad)(idx, table)
    if B_pad != B:
        out = out[:B]
    return out.reshape(bsz, hist, EMBED)


# speedup vs baseline: 4.9495x; 4.9495x over previous
import functools

import jax
import jax.numpy as jnp
from jax import lax
from jax.experimental import pallas as pl
from jax.experimental.pallas import tpu as pltpu
from jax.experimental.pallas import tpu_sc as plsc

EMBED = 32
NC, NS = 2, 16
NW = NC * NS
CHUNK = 2048


@functools.lru_cache(maxsize=None)
def _make_gather(B: int):
    per_w = B // NW
    nch = per_w // CHUNK
    mesh = plsc.VectorSubcoreMesh(core_axis_name="c", subcore_axis_name="s")

    @functools.partial(
        pl.kernel,
        mesh=mesh,
        out_type=jax.ShapeDtypeStruct((B, EMBED), jnp.float32),
        scratch_types=[
            pltpu.VMEM((CHUNK,), jnp.int32),
            pltpu.VMEM((CHUNK, EMBED), jnp.float32),
            pltpu.SemaphoreType.DMA,
        ],
        compiler_params=pltpu.CompilerParams(use_tc_tiling_on_sc=False),
    )
    def k(idx_hbm, table_hbm, out_hbm, idx_v, rows_v, sem):
        wid = lax.axis_index("s") * NC + lax.axis_index("c")
        base_w = wid * per_w

        def body(g, carry):
            base = base_w + g * CHUNK
            pltpu.sync_copy(idx_hbm.at[pl.ds(base, CHUNK)], idx_v)
            pltpu.async_copy(table_hbm.at[idx_v], rows_v, sem).wait()
            pltpu.sync_copy(rows_v, out_hbm.at[pl.ds(base, CHUNK)])
            return carry

        lax.fori_loop(0, nch, body, 0)

    return k


def kernel(input_indices, table):
    bsz, hist = input_indices.shape
    B = bsz * hist
    idx = input_indices.reshape(B).astype(jnp.int32)
    step = NW * CHUNK
    B_pad = ((B + step - 1) // step) * step
    if B_pad != B:
        idx = jnp.pad(idx, (0, B_pad - B))
    out = _make_gather(B_pad)(idx, table)
    if B_pad != B:
        out = out[:B]
    return out.reshape(bsz, hist, EMBED)


# speedup vs baseline: 4.9844x; 1.0071x over previous
import functools

import jax
import jax.numpy as jnp
from jax import lax
from jax.experimental import pallas as pl
from jax.experimental.pallas import tpu as pltpu
from jax.experimental.pallas import tpu_sc as plsc

EMBED = 32
NC, NS = 2, 16
NW = NC * NS
CHUNK = 1600


@functools.lru_cache(maxsize=None)
def _make_gather(B: int):
    per_w = B // NW
    nch = per_w // CHUNK
    assert nch % 2 == 0 and nch >= 4
    mesh = plsc.VectorSubcoreMesh(core_axis_name="c", subcore_axis_name="s")

    @functools.partial(
        pl.kernel,
        mesh=mesh,
        out_type=jax.ShapeDtypeStruct((B, EMBED), jnp.float32),
        scratch_types=[
            pltpu.VMEM((CHUNK,), jnp.int32),
            pltpu.VMEM((CHUNK,), jnp.int32),
            pltpu.VMEM((CHUNK, EMBED), jnp.float32),
            pltpu.VMEM((CHUNK, EMBED), jnp.float32),
            pltpu.SemaphoreType.DMA,
            pltpu.SemaphoreType.DMA,
            pltpu.SemaphoreType.DMA,
            pltpu.SemaphoreType.DMA,
        ],
        compiler_params=pltpu.CompilerParams(use_tc_tiling_on_sc=False),
    )
    def k(idx_hbm, table_hbm, out_hbm, idx0, idx1, rows0, rows1, g0, g1, s0, s1):
        idxs = (idx0, idx1)
        rows = (rows0, rows1)
        gsem = (g0, g1)
        ssem = (s0, s1)
        wid = lax.axis_index("s") * NC + lax.axis_index("c")
        base_w = wid * per_w

        def start_chunk(g, b):
            base = base_w + g * CHUNK
            pltpu.sync_copy(idx_hbm.at[pl.ds(base, CHUNK)], idxs[b])
            pltpu.make_async_copy(table_hbm.at[idxs[b]], rows[b], gsem[b]).start()

        def finish_chunk(g, b):
            base = base_w + g * CHUNK
            pltpu.make_async_copy(table_hbm.at[idxs[b]], rows[b], gsem[b]).wait()
            pltpu.make_async_copy(rows[b], out_hbm.at[pl.ds(base, CHUNK)], ssem[b]).start()

        def wait_store(g, b):
            base = base_w + g * CHUNK
            pltpu.make_async_copy(rows[b], out_hbm.at[pl.ds(base, CHUNK)], ssem[b]).wait()

        start_chunk(0, 0)
        start_chunk(1, 1)
        finish_chunk(0, 0)

        def body(s, carry):
            ga = 2 * s
            wait_store(ga - 2, 0)
            start_chunk(ga, 0)
            finish_chunk(ga - 1, 1)
            wait_store(ga - 1, 1)
            start_chunk(ga + 1, 1)
            finish_chunk(ga, 0)
            return carry

        lax.fori_loop(1, nch // 2, body, 0)

        wait_store(nch - 2, 0)
        finish_chunk(nch - 1, 1)
        wait_store(nch - 1, 1)

    return k


def kernel(input_indices, table):
    bsz, hist = input_indices.shape
    B = bsz * hist
    idx = input_indices.reshape(B).astype(jnp.int32)
    step = NW * CHUNK * 2
    B_pad = ((B + step - 1) // step) * step
    if B_pad != B:
        idx = jnp.pad(idx, (0, B_pad - B))
    out = _make_gather(B_pad)(idx, table)
    if B_pad != B:
        out = out[:B]
    return out.reshape(bsz, hist, EMBED)
